# baseline (speedup 1.0000x reference)
import functools

import jax
import jax.numpy as jnp
from jax import lax
from jax.experimental import pallas as pl
from jax.experimental.pallas import tpu as pltpu
from jax.experimental.pallas import tpu_sc as plsc

N = 10000
E = 160000
NODE_TYPES = 100
FEAT = 28
H = 32
B = 64
MP_ITER = 3
S2S_ITER = 4

NB = 400
NP = N
NGRID = NP // NB
CH = 128
EROWS = E // CH
CROWS = EROWS // 2
WCHA = 39
WCHB = 40
NROWS = 10240
ZR = NROWS // 16

_f32 = jnp.float32


def _tflat_body(we1_ref, be1_ref, we2_ref, be2_ref, out_ref):
    a = jnp.maximum(we1_ref[...] + be1_ref[...], 0.0)
    out_ref[...] = (
        jnp.dot(a, we2_ref[...], preferred_element_type=_f32) + be2_ref[...]
    )


def _tflat_call(W_e1, b_e1, W_e2, b_e2):
    return pl.pallas_call(
        _tflat_body,
        out_shape=jax.ShapeDtypeStruct((16, H * H), _f32),
    )(W_e1, b_e1.reshape(1, H), W_e2, b_e2.reshape(1, H * H))


def _pack4d(ht):
    return jnp.transpose(ht.reshape(NB // 8, 8, 4, 128), (0, 2, 1, 3))


def _prep_body(nt_ref, feat_ref, wemb_ref, wf_ref, bemb_ref, tcat_ref,
               h_ref, ht_ref):
    nt = nt_ref[...]
    iot = lax.broadcasted_iota(jnp.int32, (NB, 128), 1)
    oh = (iot == nt).astype(_f32)
    h = (jnp.dot(oh, wemb_ref[...], preferred_element_type=_f32)
         + jnp.dot(feat_ref[...], wf_ref[...], preferred_element_type=_f32)
         + bemb_ref[...])
    h_ref[...] = h
    ht_ref[...] = _pack4d(jnp.dot(h, tcat_ref[...],
                                  preferred_element_type=_f32))


def _prep_call(nt_p, feat_p, W_emb, W_f, b_emb, Tcat):
    return pl.pallas_call(
        _prep_body,
        grid=(NGRID,),
        in_specs=[
            pl.BlockSpec((NB, 1), lambda i: (i, 0)),
            pl.BlockSpec((NB, FEAT), lambda i: (i, 0)),
            pl.BlockSpec((128, H), lambda i: (0, 0)),
            pl.BlockSpec((FEAT, H), lambda i: (0, 0)),
            pl.BlockSpec((1, H), lambda i: (0, 0)),
            pl.BlockSpec((H, 16 * H), lambda i: (0, 0)),
        ],
        out_specs=[
            pl.BlockSpec((NB, H), lambda i: (i, 0)),
            pl.BlockSpec((NB // 8, 4, 8, 128), lambda i: (i, 0, 0, 0)),
        ],
        out_shape=[
            jax.ShapeDtypeStruct((NP, H), _f32),
            jax.ShapeDtypeStruct((NP // 8, 4, 8, 128), _f32),
        ],
    )(nt_p, feat_p, W_emb, W_f, b_emb, Tcat)


def _step_body(agg_ref, h_ref, wr_ref, bc_ref, tcat_ref, hn_ref, ht_ref):
    a = agg_ref[0] + agg_ref[1]
    hn = a + jnp.dot(h_ref[...], wr_ref[...], preferred_element_type=_f32) \
        + bc_ref[...]
    hn_ref[...] = hn
    ht_ref[...] = _pack4d(jnp.dot(hn, tcat_ref[...],
                                  preferred_element_type=_f32))


def _step_call(agg, h, Wr, bc, Tcat):
    return pl.pallas_call(
        _step_body,
        grid=(NGRID,),
        in_specs=[
            pl.BlockSpec((2, NB, H), lambda i: (0, i, 0)),
            pl.BlockSpec((NB, H), lambda i: (i, 0)),
            pl.BlockSpec((H, H), lambda i: (0, 0)),
            pl.BlockSpec((1, H), lambda i: (0, 0)),
            pl.BlockSpec((H, 16 * H), lambda i: (0, 0)),
        ],
        out_specs=[
            pl.BlockSpec((NB, H), lambda i: (i, 0)),
            pl.BlockSpec((NB // 8, 4, 8, 128), lambda i: (i, 0, 0, 0)),
        ],
        out_shape=[
            jax.ShapeDtypeStruct((NP, H), _f32),
            jax.ShapeDtypeStruct((NP // 8, 4, 8, 128), _f32),
        ],
    )(agg, h, Wr, bc, Tcat)


def _s2s_body(agg_ref, hp_ref, wr_ref, bc_ref, batch_ref,
              wih_ref, whh_ref, bih_ref, bhh_ref,
              wo1_ref, bo1_ref, wo2_ref, bo2_ref, out_ref):
    h = (agg_ref[0, :NP] + agg_ref[1, :NP]
         + jnp.dot(hp_ref[...], wr_ref[...], preferred_element_type=_f32)
         + bc_ref[...])
    h_t = jnp.transpose(h)
    bt = batch_ref[...]
    msk = lax.broadcasted_iota(jnp.int32, (B, NP), 0) == bt
    q_star = jnp.zeros((B, 2 * H), _f32)
    hs = jnp.zeros((B, H), _f32)
    cs = jnp.zeros((B, H), _f32)
    for _ in range(S2S_ITER):
        gates = (jnp.dot(q_star, wih_ref[...], preferred_element_type=_f32)
                 + bih_ref[...]
                 + jnp.dot(hs, whh_ref[...], preferred_element_type=_f32)
                 + bhh_ref[...])
        i_g = jax.nn.sigmoid(gates[:, 0:H])
        f_g = jax.nn.sigmoid(gates[:, H:2 * H])
        g_g = jnp.tanh(gates[:, 2 * H:3 * H])
        o_g = jax.nn.sigmoid(gates[:, 3 * H:4 * H])
        cs = f_g * cs + i_g * g_g
        hs = o_g * jnp.tanh(cs)
        q = hs
        s_mat = jnp.dot(q, h_t, preferred_element_type=_f32)
        sm = jnp.where(msk, s_mat, -1e30)
        m = jnp.max(sm, axis=1, keepdims=True)
        m = jnp.where(m > -1e29, m, 0.0)
        p = jnp.exp(jnp.where(msk, s_mat - m, -1e4))
        denom = jnp.sum(p, axis=1, keepdims=True)
        a = p / (denom + 1e-16)
        r = jnp.dot(a, h, preferred_element_type=_f32)
        q_star = jnp.concatenate([q, r], axis=1)
    o1 = jnp.maximum(
        jnp.dot(q_star, wo1_ref[...], preferred_element_type=_f32)
        + bo1_ref[...], 0.0)
    out_ref[...] = jnp.dot(o1, wo2_ref[...], preferred_element_type=_f32) \
        + bo2_ref[...]


def _s2s_call(agg, h, Wr, bc, batch_p, W_ihT, W_hhT, b_ih, b_hh,
              W_o1, b_o1, W_o2, b_o2):
    return pl.pallas_call(
        _s2s_body,
        out_shape=jax.ShapeDtypeStruct((B, 1), _f32),
    )(agg, h, Wr, bc, batch_p, W_ihT, W_hhT,
      b_ih.reshape(1, 4 * H), b_hh.reshape(1, 4 * H),
      W_o1, b_o1.reshape(1, H), W_o2, b_o2.reshape(1, 1))


_NBUF = 4


def _mp_sc_body(src_ref, et_ref, dst_ref, ht_ref, out_ref,
                srcv, etv, dstv, idxv, rows0, rows1, rows2, rows3, zbuf,
                aggsh, gs0, gs1, gs2, gs3, ss0, ss1, ss2, ss3):
    rows = (rows0, rows1, rows2, rows3)
    gsem = (gs0, gs1, gs2, gs3)
    ssem = (ss0, ss1, ss2, ss3)
    c = lax.axis_index("c")
    s = lax.axis_index("s")

    z16 = jnp.zeros((16,), _f32)

    def zrow(k, carry):
        for b in range(4):
            zbuf[k * 4 + b, pl.ds(0, 16)] = z16
            zbuf[k * 4 + b, pl.ds(16, 16)] = z16
        return carry

    lax.fori_loop(0, ZR // 4, zrow, 0)
    pltpu.sync_copy(zbuf, aggsh.at[pl.ds(s * ZR, ZR)])

    def pipeline(off, wch, nbuf):
        pltpu.sync_copy(src_ref.at[pl.ds(off, wch)], srcv.at[pl.ds(0, wch)])
        pltpu.sync_copy(et_ref.at[pl.ds(off, wch)], etv.at[pl.ds(0, wch)])
        pltpu.sync_copy(dst_ref.at[pl.ds(off, wch)], dstv.at[pl.ds(0, wch)])

        def idxrow(k, carry):
            for j in range(8):
                sl = pl.ds(j * 16, 16)
                sv = srcv[k, sl]
                ev = etv[k, sl]
                idxv[k, sl] = ((sv >> 3) << 7) + ((ev >> 2) << 5) \
                    + ((sv & 7) << 2) + (ev & 3)
            return carry

        lax.fori_loop(0, wch, idxrow, 0)

        for b in range(nbuf):
            pltpu.async_copy(ht_ref.at[idxv.at[b]], rows[b], gsem[b])

        def outer(kk, carry):
            for b in range(nbuf):
                k = kk * nbuf + b
                pltpu.make_async_copy(ht_ref.at[idxv.at[k]], rows[b],
                                      gsem[b]).wait()
                pltpu.async_copy(rows[b], aggsh.at[dstv.at[k]], ssem[b],
                                 add=True)

                @pl.when(kk < wch // nbuf - 1)
                def _():
                    pltpu.make_async_copy(rows[b], aggsh.at[dstv.at[k]],
                                          ssem[b]).wait()
                    pltpu.async_copy(ht_ref.at[idxv.at[k + nbuf]], rows[b],
                                     gsem[b])
            return carry

        lax.fori_loop(0, wch // nbuf, outer, 0)
        for b in range(nbuf):
            k = wch - nbuf + b
            pltpu.make_async_copy(rows[b], aggsh.at[dstv.at[k]],
                                  ssem[b]).wait()

    base = c * CROWS

    @pl.when(s < 15)
    def _():
        pipeline(base + s * WCHA, WCHA, 3)

    @pl.when(s == 15)
    def _():
        pipeline(base + 15 * WCHA, WCHB, 4)

    plsc.subcore_barrier()

    pltpu.sync_copy(aggsh.at[pl.ds(s * ZR, ZR)], zbuf)
    pltpu.sync_copy(zbuf, out_ref.at[c, pl.ds(s * ZR, ZR)])


@functools.cache
def _mp_sc_call():
    mesh = plsc.VectorSubcoreMesh(core_axis_name="c", subcore_axis_name="s",
                                  num_cores=2, num_subcores=16)
    return pl.kernel(
        _mp_sc_body,
        out_type=jax.ShapeDtypeStruct((2, NROWS, H), _f32),
        mesh=mesh,
        compiler_params=pltpu.CompilerParams(use_tc_tiling_on_sc=False),
        scratch_types=[
            pltpu.VMEM((WCHB, CH), jnp.int32),
            pltpu.VMEM((WCHB, CH), jnp.int32),
            pltpu.VMEM((WCHB, CH), jnp.int32),
            pltpu.VMEM((WCHB, CH), jnp.int32),
            pltpu.VMEM((CH, H), _f32),
            pltpu.VMEM((CH, H), _f32),
            pltpu.VMEM((CH, H), _f32),
            pltpu.VMEM((CH, H), _f32),
            pltpu.VMEM((ZR, H), _f32),
            pltpu.VMEM_SHARED((NROWS, H), _f32),
            pltpu.SemaphoreType.DMA,
            pltpu.SemaphoreType.DMA,
            pltpu.SemaphoreType.DMA,
            pltpu.SemaphoreType.DMA,
            pltpu.SemaphoreType.DMA,
            pltpu.SemaphoreType.DMA,
            pltpu.SemaphoreType.DMA,
            pltpu.SemaphoreType.DMA,
        ],
    )


def kernel(node_type, node_feat, edge_index, edge_type, batch,
           W_emb, b_emb, W_e1, b_e1, W_e2, b_e2, W_root, b_conv,
           W_ih, W_hh, b_ih, b_hh, W_o1, b_o1, W_o2, b_o2):
    i32 = jnp.int32
    nt_p = node_type.astype(i32).reshape(NP, 1)
    W_f = W_emb[NODE_TYPES:]
    batch_p = batch.astype(i32).reshape(1, NP)

    src2 = edge_index[0].astype(i32).reshape(EROWS, CH)
    et2 = edge_type.astype(i32).reshape(EROWS, CH)
    dst2 = edge_index[1].astype(i32).reshape(EROWS, CH)

    tflat = _tflat_call(W_e1, b_e1, W_e2, b_e2)
    tcat = tflat.reshape(16, H, H).transpose(1, 0, 2).reshape(H, 16 * H)

    h, htcat = _prep_call(nt_p, node_feat, W_emb, W_f, b_emb.reshape(1, H), tcat)
    for i in range(MP_ITER - 1):
        agg = _mp_sc_call()(src2, et2, dst2, htcat.reshape(NP * 16, H))
        h, htcat = _step_call(agg, h, W_root[i], b_conv[i].reshape(1, H), tcat)
    agg = _mp_sc_call()(src2, et2, dst2, htcat.reshape(NP * 16, H))

    return _s2s_call(agg, h, W_root[MP_ITER - 1],
                     b_conv[MP_ITER - 1].reshape(1, H), batch_p,
                     W_ih.T, W_hh.T, b_ih, b_hh, W_o1, b_o1, W_o2, b_o2)


# speedup vs baseline: 1.0659x; 1.0659x over previous
import functools

import jax
import jax.numpy as jnp
from jax import lax
from jax.experimental import pallas as pl
from jax.experimental.pallas import tpu as pltpu
from jax.experimental.pallas import tpu_sc as plsc

N = 10000
E = 160000
NODE_TYPES = 100
FEAT = 28
H = 32
B = 64
MP_ITER = 3
S2S_ITER = 4

NB = 512
NP = 10240
NGRID = NP // NB
EP = 163840
WCH = 40
CH = 128
EROWS = EP // CH
NROWS = NP + 128
ZR = NROWS // 16

_f32 = jnp.float32


def _tflat_body(we1_ref, be1_ref, we2_ref, be2_ref, out_ref):
    a = jnp.maximum(we1_ref[...] + be1_ref[...], 0.0)
    out_ref[...] = (
        jnp.dot(a, we2_ref[...], preferred_element_type=_f32) + be2_ref[...]
    )


def _tflat_call(W_e1, b_e1, W_e2, b_e2):
    return pl.pallas_call(
        _tflat_body,
        out_shape=jax.ShapeDtypeStruct((16, H * H), _f32),
    )(W_e1, b_e1.reshape(1, H), W_e2, b_e2.reshape(1, H * H))


def _pack4d(ht):
    return jnp.transpose(ht.reshape(NB // 8, 8, 4, 128), (0, 2, 1, 3))


def _prep_body(nt_ref, feat_ref, wemb_ref, wf_ref, bemb_ref, tcat_ref,
               h_ref, ht_ref):
    nt = nt_ref[...]
    iot = lax.broadcasted_iota(jnp.int32, (NB, 128), 1)
    oh = (iot == nt).astype(_f32)
    h = (jnp.dot(oh, wemb_ref[...], preferred_element_type=_f32)
         + jnp.dot(feat_ref[...], wf_ref[...], preferred_element_type=_f32)
         + bemb_ref[...])
    h_ref[...] = h
    ht_ref[...] = _pack4d(jnp.dot(h, tcat_ref[...],
                                  preferred_element_type=_f32))


def _prep_call(nt_p, feat_p, W_emb, W_f, b_emb, Tcat):
    return pl.pallas_call(
        _prep_body,
        grid=(NGRID,),
        in_specs=[
            pl.BlockSpec((NB, 1), lambda i: (i, 0)),
            pl.BlockSpec((NB, H), lambda i: (i, 0)),
            pl.BlockSpec((128, H), lambda i: (0, 0)),
            pl.BlockSpec((H, H), lambda i: (0, 0)),
            pl.BlockSpec((1, H), lambda i: (0, 0)),
            pl.BlockSpec((H, 16 * H), lambda i: (0, 0)),
        ],
        out_specs=[
            pl.BlockSpec((NB, H), lambda i: (i, 0)),
            pl.BlockSpec((NB // 8, 4, 8, 128), lambda i: (i, 0, 0, 0)),
        ],
        out_shape=[
            jax.ShapeDtypeStruct((NP, H), _f32),
            jax.ShapeDtypeStruct((NP // 8, 4, 8, 128), _f32),
        ],
    )(nt_p, feat_p, W_emb, W_f, b_emb, Tcat)


def _step_body(agg_ref, h_ref, wr_ref, bc_ref, tcat_ref, hn_ref, ht_ref):
    a = agg_ref[0] + agg_ref[1]
    hn = a + jnp.dot(h_ref[...], wr_ref[...], preferred_element_type=_f32) \
        + bc_ref[...]
    hn_ref[...] = hn
    ht_ref[...] = _pack4d(jnp.dot(hn, tcat_ref[...],
                                  preferred_element_type=_f32))


def _step_call(agg, h, Wr, bc, Tcat):
    return pl.pallas_call(
        _step_body,
        grid=(NGRID,),
        in_specs=[
            pl.BlockSpec((2, NB, H), lambda i: (0, i, 0)),
            pl.BlockSpec((NB, H), lambda i: (i, 0)),
            pl.BlockSpec((H, H), lambda i: (0, 0)),
            pl.BlockSpec((1, H), lambda i: (0, 0)),
            pl.BlockSpec((H, 16 * H), lambda i: (0, 0)),
        ],
        out_specs=[
            pl.BlockSpec((NB, H), lambda i: (i, 0)),
            pl.BlockSpec((NB // 8, 4, 8, 128), lambda i: (i, 0, 0, 0)),
        ],
        out_shape=[
            jax.ShapeDtypeStruct((NP, H), _f32),
            jax.ShapeDtypeStruct((NP // 8, 4, 8, 128), _f32),
        ],
    )(agg, h, Wr, bc, Tcat)


def _s2s_body(agg_ref, hp_ref, wr_ref, bc_ref, batch_ref,
              wih_ref, whh_ref, bih_ref, bhh_ref,
              wo1_ref, bo1_ref, wo2_ref, bo2_ref, out_ref):
    h = (agg_ref[0, :NP] + agg_ref[1, :NP]
         + jnp.dot(hp_ref[...], wr_ref[...], preferred_element_type=_f32)
         + bc_ref[...])
    h_t = jnp.transpose(h)
    bt = batch_ref[...]
    msk = lax.broadcasted_iota(jnp.int32, (B, NP), 0) == bt
    q_star = jnp.zeros((B, 2 * H), _f32)
    hs = jnp.zeros((B, H), _f32)
    cs = jnp.zeros((B, H), _f32)
    for _ in range(S2S_ITER):
        gates = (jnp.dot(q_star, wih_ref[...], preferred_element_type=_f32)
                 + bih_ref[...]
                 + jnp.dot(hs, whh_ref[...], preferred_element_type=_f32)
                 + bhh_ref[...])
        i_g = jax.nn.sigmoid(gates[:, 0:H])
        f_g = jax.nn.sigmoid(gates[:, H:2 * H])
        g_g = jnp.tanh(gates[:, 2 * H:3 * H])
        o_g = jax.nn.sigmoid(gates[:, 3 * H:4 * H])
        cs = f_g * cs + i_g * g_g
        hs = o_g * jnp.tanh(cs)
        q = hs
        s_mat = jnp.dot(q, h_t, preferred_element_type=_f32)
        sm = jnp.where(msk, s_mat, -1e30)
        m = jnp.max(sm, axis=1, keepdims=True)
        m = jnp.where(m > -1e29, m, 0.0)
        p = jnp.exp(jnp.where(msk, s_mat - m, -1e4))
        denom = jnp.sum(p, axis=1, keepdims=True)
        a = p / (denom + 1e-16)
        r = jnp.dot(a, h, preferred_element_type=_f32)
        q_star = jnp.concatenate([q, r], axis=1)
    o1 = jnp.maximum(
        jnp.dot(q_star, wo1_ref[...], preferred_element_type=_f32)
        + bo1_ref[...], 0.0)
    out_ref[...] = jnp.dot(o1, wo2_ref[...], preferred_element_type=_f32) \
        + bo2_ref[...]


def _s2s_call(agg, h, Wr, bc, batch_p, W_ihT, W_hhT, b_ih, b_hh,
              W_o1, b_o1, W_o2, b_o2):
    return pl.pallas_call(
        _s2s_body,
        out_shape=jax.ShapeDtypeStruct((B, 1), _f32),
    )(agg, h, Wr, bc, batch_p, W_ihT, W_hhT,
      b_ih.reshape(1, 4 * H), b_hh.reshape(1, 4 * H),
      W_o1, b_o1.reshape(1, H), W_o2, b_o2.reshape(1, 1))


_NBUF = 4


def _mp_sc_body(src_ref, et_ref, dst_ref, ht_ref, out_ref,
                srcv, etv, dstv, idxv, rows0, rows1, rows2, rows3, zbuf,
                aggsh, gs0, gs1, gs2, gs3, ss0, ss1, ss2, ss3):
    rows = (rows0, rows1, rows2, rows3)
    gsem = (gs0, gs1, gs2, gs3)
    ssem = (ss0, ss1, ss2, ss3)
    c = lax.axis_index("c")
    s = lax.axis_index("s")

    z16 = jnp.zeros((16,), _f32)

    def zrow(k, carry):
        for b in range(4):
            zbuf[k * 4 + b, pl.ds(0, 16)] = z16
            zbuf[k * 4 + b, pl.ds(16, 16)] = z16
        return carry

    lax.fori_loop(0, ZR // 4, zrow, 0)
    pltpu.sync_copy(zbuf, aggsh.at[pl.ds(s * ZR, ZR)])

    def pipeline(off, wch, nbuf):
        pltpu.sync_copy(src_ref.at[pl.ds(off, wch)], srcv.at[pl.ds(0, wch)])
        pltpu.sync_copy(et_ref.at[pl.ds(off, wch)], etv.at[pl.ds(0, wch)])
        pltpu.sync_copy(dst_ref.at[pl.ds(off, wch)], dstv.at[pl.ds(0, wch)])

        def idxrow(k, carry):
            for j in range(8):
                sl = pl.ds(j * 16, 16)
                sv = srcv[k, sl]
                ev = etv[k, sl]
                idxv[k, sl] = ((sv >> 3) << 7) + ((ev >> 2) << 5) \
                    + ((sv & 7) << 2) + (ev & 3)
            return carry

        lax.fori_loop(0, wch, idxrow, 0)

        for b in range(nbuf):
            pltpu.async_copy(ht_ref.at[idxv.at[b]], rows[b], gsem[b])

        def outer(kk, carry):
            for b in range(nbuf):
                k = kk * nbuf + b
                pltpu.make_async_copy(ht_ref.at[idxv.at[k]], rows[b],
                                      gsem[b]).wait()
                pltpu.async_copy(rows[b], aggsh.at[dstv.at[k]], ssem[b],
                                 add=True)

                @pl.when(kk < wch // nbuf - 1)
                def _():
                    pltpu.make_async_copy(rows[b], aggsh.at[dstv.at[k]],
                                          ssem[b]).wait()
                    pltpu.async_copy(ht_ref.at[idxv.at[k + nbuf]], rows[b],
                                     gsem[b])
            return carry

        lax.fori_loop(0, wch // nbuf, outer, 0)
        for b in range(nbuf):
            k = wch - nbuf + b
            pltpu.make_async_copy(rows[b], aggsh.at[dstv.at[k]],
                                  ssem[b]).wait()

    w = c * 16 + s
    pipeline(w * WCH, WCH, _NBUF)

    plsc.subcore_barrier()

    pltpu.sync_copy(aggsh.at[pl.ds(s * ZR, ZR)], zbuf)
    pltpu.sync_copy(zbuf, out_ref.at[c, pl.ds(s * ZR, ZR)])


@functools.cache
def _mp_sc_call():
    mesh = plsc.VectorSubcoreMesh(core_axis_name="c", subcore_axis_name="s",
                                  num_cores=2, num_subcores=16)
    return pl.kernel(
        _mp_sc_body,
        out_type=jax.ShapeDtypeStruct((2, NROWS, H), _f32),
        mesh=mesh,
        compiler_params=pltpu.CompilerParams(use_tc_tiling_on_sc=False),
        scratch_types=[
            pltpu.VMEM((WCH, CH), jnp.int32),
            pltpu.VMEM((WCH, CH), jnp.int32),
            pltpu.VMEM((WCH, CH), jnp.int32),
            pltpu.VMEM((WCH, CH), jnp.int32),
            pltpu.VMEM((CH, H), _f32),
            pltpu.VMEM((CH, H), _f32),
            pltpu.VMEM((CH, H), _f32),
            pltpu.VMEM((CH, H), _f32),
            pltpu.VMEM((ZR, H), _f32),
            pltpu.VMEM_SHARED((NROWS, H), _f32),
            pltpu.SemaphoreType.DMA,
            pltpu.SemaphoreType.DMA,
            pltpu.SemaphoreType.DMA,
            pltpu.SemaphoreType.DMA,
            pltpu.SemaphoreType.DMA,
            pltpu.SemaphoreType.DMA,
            pltpu.SemaphoreType.DMA,
            pltpu.SemaphoreType.DMA,
        ],
    )


def kernel(node_type, node_feat, edge_index, edge_type, batch,
           W_emb, b_emb, W_e1, b_e1, W_e2, b_e2, W_root, b_conv,
           W_ih, W_hh, b_ih, b_hh, W_o1, b_o1, W_o2, b_o2):
    i32 = jnp.int32
    nt_p = jnp.pad(node_type.astype(i32), (0, NP - N)).reshape(NP, 1)
    feat_p = jnp.pad(node_feat, ((0, NP - N), (0, H - FEAT)))
    W_f = jnp.pad(W_emb[NODE_TYPES:], ((0, H - FEAT), (0, 0)))
    batch_p = jnp.pad(batch.astype(i32), (0, NP - N),
                      constant_values=127).reshape(1, NP)

    pad_ar = jnp.arange(EP - E, dtype=i32)
    src2 = jnp.concatenate([edge_index[0].astype(i32),
                            pad_ar % N]).reshape(EROWS, CH)
    et2 = jnp.pad(edge_type.astype(i32), (0, EP - E)).reshape(EROWS, CH)
    pad_dst = NP + (pad_ar % (NROWS - NP))
    dst2 = jnp.concatenate([edge_index[1].astype(i32),
                            pad_dst]).reshape(EROWS, CH)

    tflat = _tflat_call(W_e1, b_e1, W_e2, b_e2)
    tcat = tflat.reshape(16, H, H).transpose(1, 0, 2).reshape(H, 16 * H)

    h, htcat = _prep_call(nt_p, feat_p, W_emb, W_f, b_emb.reshape(1, H), tcat)
    for i in range(MP_ITER - 1):
        agg = _mp_sc_call()(src2, et2, dst2, htcat.reshape(NP * 16, H))
        h, htcat = _step_call(agg, h, W_root[i], b_conv[i].reshape(1, H), tcat)
    agg = _mp_sc_call()(src2, et2, dst2, htcat.reshape(NP * 16, H))

    return _s2s_call(agg, h, W_root[MP_ITER - 1],
                     b_conv[MP_ITER - 1].reshape(1, H), batch_p,
                     W_ih.T, W_hh.T, b_ih, b_hh, W_o1, b_o1, W_o2, b_o2)


# speedup vs baseline: 1.1236x; 1.0542x over previous
import functools

import jax
import jax.numpy as jnp
from jax import lax
from jax.experimental import pallas as pl
from jax.experimental.pallas import tpu as pltpu
from jax.experimental.pallas import tpu_sc as plsc

N = 10000
E = 160000
NODE_TYPES = 100
FEAT = 28
H = 32
B = 64
MP_ITER = 3
S2S_ITER = 4

NB = 512
NP = 10240
NGRID = NP // NB
EP = 163840
WCH = 40
CH = 128
EROWS = EP // CH
NROWS = NP + 128
ZR = NROWS // 16

_f32 = jnp.float32


def _tflat_body(we1_ref, be1_ref, we2_ref, be2_ref, out_ref):
    a = jnp.maximum(we1_ref[...] + be1_ref[...], 0.0)
    out_ref[...] = (
        jnp.dot(a, we2_ref[...], preferred_element_type=_f32) + be2_ref[...]
    )


def _tflat_call(W_e1, b_e1, W_e2, b_e2):
    return pl.pallas_call(
        _tflat_body,
        out_shape=jax.ShapeDtypeStruct((16, H * H), _f32),
    )(W_e1, b_e1.reshape(1, H), W_e2, b_e2.reshape(1, H * H))


def _pack4d(ht):
    return jnp.transpose(ht.reshape(NB // 8, 8, 4, 128), (0, 2, 1, 3))


def _prep_body(nt_ref, feat_ref, wemb_ref, wf_ref, bemb_ref, tcat_ref,
               h_ref, ht_ref):
    nt = nt_ref[...]
    iot = lax.broadcasted_iota(jnp.int32, (NB, 128), 1)
    oh = (iot == nt).astype(_f32)
    h = (jnp.dot(oh, wemb_ref[...], preferred_element_type=_f32)
         + jnp.dot(feat_ref[...], wf_ref[...], preferred_element_type=_f32)
         + bemb_ref[...])
    h_ref[...] = h
    ht_ref[...] = _pack4d(jnp.dot(h, tcat_ref[...],
                                  preferred_element_type=_f32))


def _prep_call(nt_p, feat_p, W_emb, W_f, b_emb, Tcat):
    return pl.pallas_call(
        _prep_body,
        grid=(NGRID,),
        in_specs=[
            pl.BlockSpec((NB, 1), lambda i: (i, 0)),
            pl.BlockSpec((NB, H), lambda i: (i, 0)),
            pl.BlockSpec((128, H), lambda i: (0, 0)),
            pl.BlockSpec((H, H), lambda i: (0, 0)),
            pl.BlockSpec((1, H), lambda i: (0, 0)),
            pl.BlockSpec((H, 16 * H), lambda i: (0, 0)),
        ],
        out_specs=[
            pl.BlockSpec((NB, H), lambda i: (i, 0)),
            pl.BlockSpec((NB // 8, 4, 8, 128), lambda i: (i, 0, 0, 0)),
        ],
        out_shape=[
            jax.ShapeDtypeStruct((NP, H), _f32),
            jax.ShapeDtypeStruct((NP // 8, 4, 8, 128), _f32),
        ],
    )(nt_p, feat_p, W_emb, W_f, b_emb, Tcat)


def _step_body(agg_ref, h_ref, wr_ref, bc_ref, tcat_ref, hn_ref, ht_ref):
    a = agg_ref[0] + agg_ref[1]
    hn = a + jnp.dot(h_ref[...], wr_ref[...], preferred_element_type=_f32) \
        + bc_ref[...]
    hn_ref[...] = hn
    ht_ref[...] = _pack4d(jnp.dot(hn, tcat_ref[...],
                                  preferred_element_type=_f32))


def _step_call(agg, h, Wr, bc, Tcat):
    return pl.pallas_call(
        _step_body,
        grid=(NGRID,),
        in_specs=[
            pl.BlockSpec((2, NB, H), lambda i: (0, i, 0)),
            pl.BlockSpec((NB, H), lambda i: (i, 0)),
            pl.BlockSpec((H, H), lambda i: (0, 0)),
            pl.BlockSpec((1, H), lambda i: (0, 0)),
            pl.BlockSpec((H, 16 * H), lambda i: (0, 0)),
        ],
        out_specs=[
            pl.BlockSpec((NB, H), lambda i: (i, 0)),
            pl.BlockSpec((NB // 8, 4, 8, 128), lambda i: (i, 0, 0, 0)),
        ],
        out_shape=[
            jax.ShapeDtypeStruct((NP, H), _f32),
            jax.ShapeDtypeStruct((NP // 8, 4, 8, 128), _f32),
        ],
    )(agg, h, Wr, bc, Tcat)


def _s2s_body(agg_ref, hp_ref, wr_ref, bc_ref, batch_ref,
              wih_ref, whh_ref, bih_ref, bhh_ref,
              wo1_ref, bo1_ref, wo2_ref, bo2_ref, out_ref):
    h = (agg_ref[0, :NP] + agg_ref[1, :NP]
         + jnp.dot(hp_ref[...], wr_ref[...], preferred_element_type=_f32)
         + bc_ref[...])
    h_t = jnp.transpose(h)
    bt = batch_ref[...]
    msk = lax.broadcasted_iota(jnp.int32, (B, NP), 0) == bt
    q_star = jnp.zeros((B, 2 * H), _f32)
    hs = jnp.zeros((B, H), _f32)
    cs = jnp.zeros((B, H), _f32)
    for _ in range(S2S_ITER):
        gates = (jnp.dot(q_star, wih_ref[...], preferred_element_type=_f32)
                 + bih_ref[...]
                 + jnp.dot(hs, whh_ref[...], preferred_element_type=_f32)
                 + bhh_ref[...])
        i_g = jax.nn.sigmoid(gates[:, 0:H])
        f_g = jax.nn.sigmoid(gates[:, H:2 * H])
        g_g = jnp.tanh(gates[:, 2 * H:3 * H])
        o_g = jax.nn.sigmoid(gates[:, 3 * H:4 * H])
        cs = f_g * cs + i_g * g_g
        hs = o_g * jnp.tanh(cs)
        q = hs
        s_mat = jnp.dot(q, h_t, preferred_element_type=_f32)
        sm = jnp.where(msk, s_mat, -1e30)
        m = jnp.max(sm, axis=1, keepdims=True)
        m = jnp.where(m > -1e29, m, 0.0)
        p = jnp.exp(jnp.where(msk, s_mat - m, -1e4))
        denom = jnp.sum(p, axis=1, keepdims=True)
        a = p / (denom + 1e-16)
        r = jnp.dot(a, h, preferred_element_type=_f32)
        q_star = jnp.concatenate([q, r], axis=1)
    o1 = jnp.maximum(
        jnp.dot(q_star, wo1_ref[...], preferred_element_type=_f32)
        + bo1_ref[...], 0.0)
    out_ref[...] = jnp.dot(o1, wo2_ref[...], preferred_element_type=_f32) \
        + bo2_ref[...]


def _s2s_call(agg, h, Wr, bc, batch_p, W_ihT, W_hhT, b_ih, b_hh,
              W_o1, b_o1, W_o2, b_o2):
    return pl.pallas_call(
        _s2s_body,
        out_shape=jax.ShapeDtypeStruct((B, 1), _f32),
    )(agg, h, Wr, bc, batch_p, W_ihT, W_hhT,
      b_ih.reshape(1, 4 * H), b_hh.reshape(1, 4 * H),
      W_o1, b_o1.reshape(1, H), W_o2, b_o2.reshape(1, 1))


_NBUF = 5


def _mp_sc_body(src_ref, et_ref, dst_ref, ht_ref, out_ref,
                srcv, etv, dstv, idxv, rows0, rows1, rows2, rows3, rows4,
                zbuf, aggsh, gs0, gs1, gs2, gs3, gs4,
                ss0, ss1, ss2, ss3, ss4):
    rows = (rows0, rows1, rows2, rows3, rows4)
    gsem = (gs0, gs1, gs2, gs3, gs4)
    ssem = (ss0, ss1, ss2, ss3, ss4)
    c = lax.axis_index("c")
    s = lax.axis_index("s")

    def pipeline(off, wch, nbuf):
        pltpu.async_copy(src_ref.at[pl.ds(off * CH, wch * CH)],
                         srcv.at[pl.ds(0, wch * CH)], gs0)
        pltpu.async_copy(et_ref.at[pl.ds(off * CH, wch * CH)],
                         etv.at[pl.ds(0, wch * CH)], gs1)
        pltpu.async_copy(dst_ref.at[pl.ds(off, wch)],
                         dstv.at[pl.ds(0, wch)], gs2)

        z16 = jnp.zeros((16,), _f32)

        def zrow(k, carry):
            for b in range(4):
                zbuf[k * 4 + b, pl.ds(0, 16)] = z16
                zbuf[k * 4 + b, pl.ds(16, 16)] = z16
            return carry

        lax.fori_loop(0, ZR // 4, zrow, 0)
        pltpu.sync_copy(zbuf, aggsh.at[pl.ds(s * ZR, ZR)])

        pltpu.make_async_copy(src_ref.at[pl.ds(off * CH, wch * CH)],
                              srcv.at[pl.ds(0, wch * CH)], gs0).wait()
        pltpu.make_async_copy(et_ref.at[pl.ds(off * CH, wch * CH)],
                              etv.at[pl.ds(0, wch * CH)], gs1).wait()
        pltpu.make_async_copy(dst_ref.at[pl.ds(off, wch)],
                              dstv.at[pl.ds(0, wch)], gs2).wait()

        def idxrow(k, carry):
            for j in range(8):
                sl = pl.ds(j * 16, 16)
                sv = srcv[pl.ds(k * CH + j * 16, 16)]
                ev = etv[pl.ds(k * CH + j * 16, 16)]
                idxv[k, sl] = ((sv >> 3) << 7) + ((ev >> 2) << 5) \
                    + ((sv & 7) << 2) + (ev & 3)
            return carry

        lax.fori_loop(0, wch, idxrow, 0)

        plsc.subcore_barrier()

        for b in range(nbuf):
            pltpu.async_copy(ht_ref.at[idxv.at[b]], rows[b], gsem[b])

        def outer(kk, carry):
            for b in range(nbuf):
                k = kk * nbuf + b
                pltpu.make_async_copy(ht_ref.at[idxv.at[k]], rows[b],
                                      gsem[b]).wait()
                pltpu.async_copy(rows[b], aggsh.at[dstv.at[k]], ssem[b],
                                 add=True)

                @pl.when(kk < wch // nbuf - 1)
                def _():
                    pltpu.make_async_copy(rows[b], aggsh.at[dstv.at[k]],
                                          ssem[b]).wait()
                    pltpu.async_copy(ht_ref.at[idxv.at[k + nbuf]], rows[b],
                                     gsem[b])
            return carry

        lax.fori_loop(0, wch // nbuf, outer, 0)
        for b in range(nbuf):
            k = wch - nbuf + b
            pltpu.make_async_copy(rows[b], aggsh.at[dstv.at[k]],
                                  ssem[b]).wait()

    w = c * 16 + s
    pipeline(w * WCH, WCH, _NBUF)
    plsc.subcore_barrier()

    pltpu.sync_copy(aggsh.at[pl.ds(s * ZR, ZR)], zbuf)
    pltpu.sync_copy(zbuf, out_ref.at[c, pl.ds(s * ZR, ZR)])


@functools.cache
def _mp_sc_call():
    mesh = plsc.VectorSubcoreMesh(core_axis_name="c", subcore_axis_name="s",
                                  num_cores=2, num_subcores=16)
    return pl.kernel(
        _mp_sc_body,
        out_type=jax.ShapeDtypeStruct((2, NROWS, H), _f32),
        mesh=mesh,
        compiler_params=pltpu.CompilerParams(use_tc_tiling_on_sc=False),
        scratch_types=[
            pltpu.VMEM((WCH * CH,), jnp.int32),
            pltpu.VMEM((WCH * CH,), jnp.int32),
            pltpu.VMEM((WCH, CH), jnp.int32),
            pltpu.VMEM((WCH, CH), jnp.int32),
            pltpu.VMEM((CH, H), _f32),
            pltpu.VMEM((CH, H), _f32),
            pltpu.VMEM((CH, H), _f32),
            pltpu.VMEM((CH, H), _f32),
            pltpu.VMEM((CH, H), _f32),
            pltpu.VMEM((ZR, H), _f32),
            pltpu.VMEM_SHARED((NROWS, H), _f32),
            pltpu.SemaphoreType.DMA,
            pltpu.SemaphoreType.DMA,
            pltpu.SemaphoreType.DMA,
            pltpu.SemaphoreType.DMA,
            pltpu.SemaphoreType.DMA,
            pltpu.SemaphoreType.DMA,
            pltpu.SemaphoreType.DMA,
            pltpu.SemaphoreType.DMA,
            pltpu.SemaphoreType.DMA,
            pltpu.SemaphoreType.DMA,
        ],
    )


def kernel(node_type, node_feat, edge_index, edge_type, batch,
           W_emb, b_emb, W_e1, b_e1, W_e2, b_e2, W_root, b_conv,
           W_ih, W_hh, b_ih, b_hh, W_o1, b_o1, W_o2, b_o2):
    i32 = jnp.int32
    nt_p = jnp.pad(node_type.astype(i32), (0, NP - N)).reshape(NP, 1)
    feat_p = jnp.pad(node_feat, ((0, NP - N), (0, H - FEAT)))
    W_f = jnp.pad(W_emb[NODE_TYPES:], ((0, H - FEAT), (0, 0)))
    batch_p = jnp.pad(batch.astype(i32), (0, NP - N),
                      constant_values=127).reshape(1, NP)

    pad_ar = jnp.arange(EP - E, dtype=i32)
    src1 = jnp.concatenate([edge_index[0].astype(i32), pad_ar % N])
    et1 = jnp.pad(edge_type.astype(i32), (0, EP - E))
    pad_dst = NP + (pad_ar % (NROWS - NP))
    dst2 = jnp.concatenate([edge_index[1].astype(i32),
                            pad_dst]).reshape(EROWS, CH)

    tflat = _tflat_call(W_e1, b_e1, W_e2, b_e2)
    tcat = tflat.reshape(16, H, H).transpose(1, 0, 2).reshape(H, 16 * H)

    h, htcat = _prep_call(nt_p, feat_p, W_emb, W_f, b_emb.reshape(1, H), tcat)
    for i in range(MP_ITER - 1):
        agg = _mp_sc_call()(src1, et1, dst2, htcat.reshape(NP * 16, H))
        h, htcat = _step_call(agg, h, W_root[i], b_conv[i].reshape(1, H), tcat)
    agg = _mp_sc_call()(src1, et1, dst2, htcat.reshape(NP * 16, H))

    return _s2s_call(agg, h, W_root[MP_ITER - 1],
                     b_conv[MP_ITER - 1].reshape(1, H), batch_p,
                     W_ih.T, W_hh.T, b_ih, b_hh, W_o1, b_o1, W_o2, b_o2)


# speedup vs baseline: 1.2414x; 1.1048x over previous
import functools

import jax
import jax.numpy as jnp
from jax import lax
from jax.experimental import pallas as pl
from jax.experimental.pallas import tpu as pltpu
from jax.experimental.pallas import tpu_sc as plsc

N = 10000
E = 160000
NODE_TYPES = 100
FEAT = 28
H = 32
B = 64
MP_ITER = 3
S2S_ITER = 4

NB = 512
NP = 10240
NGRID = NP // NB
EP = 163840
WCH = 40
CH = 128
EROWS = EP // CH
NROWS = NP + 128
ZR = NROWS // 16

_f32 = jnp.float32


def _tflat_body(we1_ref, be1_ref, we2_ref, be2_ref, out_ref):
    a = jnp.maximum(we1_ref[...] + be1_ref[...], 0.0)
    out_ref[...] = (
        jnp.dot(a, we2_ref[...], preferred_element_type=_f32) + be2_ref[...]
    )


def _tflat_call(W_e1, b_e1, W_e2, b_e2):
    return pl.pallas_call(
        _tflat_body,
        out_shape=jax.ShapeDtypeStruct((16, H * H), _f32),
    )(W_e1, b_e1.reshape(1, H), W_e2, b_e2.reshape(1, H * H))


def _pack4d(ht):
    return jnp.transpose(ht.reshape(NB // 8, 8, 4, 128), (0, 2, 1, 3))


def _prep_body(nt_ref, feat_ref, wemb_ref, wf_ref, bemb_ref, tcat_ref,
               h_ref, ht_ref):
    nt = nt_ref[...]
    iot = lax.broadcasted_iota(jnp.int32, (NB, 128), 1)
    oh = (iot == nt).astype(_f32)
    h = (jnp.dot(oh, wemb_ref[...], preferred_element_type=_f32)
         + jnp.dot(feat_ref[...], wf_ref[...], preferred_element_type=_f32)
         + bemb_ref[...])
    h_ref[...] = h
    ht_ref[...] = _pack4d(jnp.dot(h, tcat_ref[...],
                                  preferred_element_type=_f32))


def _prep_call(nt_p, feat_p, W_emb, W_f, b_emb, Tcat):
    return pl.pallas_call(
        _prep_body,
        grid=(NGRID,),
        in_specs=[
            pl.BlockSpec((NB, 1), lambda i: (i, 0)),
            pl.BlockSpec((NB, H), lambda i: (i, 0)),
            pl.BlockSpec((128, H), lambda i: (0, 0)),
            pl.BlockSpec((H, H), lambda i: (0, 0)),
            pl.BlockSpec((1, H), lambda i: (0, 0)),
            pl.BlockSpec((H, 16 * H), lambda i: (0, 0)),
        ],
        out_specs=[
            pl.BlockSpec((NB, H), lambda i: (i, 0)),
            pl.BlockSpec((NB // 8, 4, 8, 128), lambda i: (i, 0, 0, 0)),
        ],
        out_shape=[
            jax.ShapeDtypeStruct((NP, H), _f32),
            jax.ShapeDtypeStruct((NP // 8, 4, 8, 128), _f32),
        ],
    )(nt_p, feat_p, W_emb, W_f, b_emb, Tcat)


def _step_body(agg_ref, h_ref, wr_ref, bc_ref, tcat_ref, hn_ref, ht_ref):
    a2 = agg_ref[0] + agg_ref[1]
    rid = lax.broadcasted_iota(jnp.int32, (NB, NB // 4), 0) // 4
    uid = lax.broadcasted_iota(jnp.int32, (NB, NB // 4), 1)
    rowsel = jnp.dot((rid == uid).astype(_f32), a2,
                     preferred_element_type=_f32)
    lb = lax.broadcasted_iota(jnp.int32, (NB, 1), 0) % 4
    a = (jnp.where(lb == 0, rowsel[:, 0:H], 0.0)
         + jnp.where(lb == 1, rowsel[:, H:2 * H], 0.0)
         + jnp.where(lb == 2, rowsel[:, 2 * H:3 * H], 0.0)
         + jnp.where(lb == 3, rowsel[:, 3 * H:4 * H], 0.0))
    hn = a + jnp.dot(h_ref[...], wr_ref[...], preferred_element_type=_f32) \
        + bc_ref[...]
    hn_ref[...] = hn
    ht_ref[...] = _pack4d(jnp.dot(hn, tcat_ref[...],
                                  preferred_element_type=_f32))


def _step_call(agg, h, Wr, bc, Tcat):
    return pl.pallas_call(
        _step_body,
        grid=(NGRID,),
        in_specs=[
            pl.BlockSpec((2, NB * H // 128, 128), lambda i: (0, i, 0)),
            pl.BlockSpec((NB, H), lambda i: (i, 0)),
            pl.BlockSpec((H, H), lambda i: (0, 0)),
            pl.BlockSpec((1, H), lambda i: (0, 0)),
            pl.BlockSpec((H, 16 * H), lambda i: (0, 0)),
        ],
        out_specs=[
            pl.BlockSpec((NB, H), lambda i: (i, 0)),
            pl.BlockSpec((NB // 8, 4, 8, 128), lambda i: (i, 0, 0, 0)),
        ],
        out_shape=[
            jax.ShapeDtypeStruct((NP, H), _f32),
            jax.ShapeDtypeStruct((NP // 8, 4, 8, 128), _f32),
        ],
    )(agg, h, Wr, bc, Tcat)


def _s2s_body(agg_ref, hp_ref, wr_ref, bc_ref, batch_ref,
              wih_ref, whh_ref, bih_ref, bhh_ref,
              wo1_ref, bo1_ref, wo2_ref, bo2_ref, out_ref):
    h = (agg_ref[0, :NP] + agg_ref[1, :NP]
         + jnp.dot(hp_ref[...], wr_ref[...], preferred_element_type=_f32)
         + bc_ref[...])
    h_t = jnp.transpose(h)
    bt = batch_ref[...]
    msk = lax.broadcasted_iota(jnp.int32, (B, NP), 0) == bt
    q_star = jnp.zeros((B, 2 * H), _f32)
    hs = jnp.zeros((B, H), _f32)
    cs = jnp.zeros((B, H), _f32)
    for _ in range(S2S_ITER):
        gates = (jnp.dot(q_star, wih_ref[...], preferred_element_type=_f32)
                 + bih_ref[...]
                 + jnp.dot(hs, whh_ref[...], preferred_element_type=_f32)
                 + bhh_ref[...])
        i_g = jax.nn.sigmoid(gates[:, 0:H])
        f_g = jax.nn.sigmoid(gates[:, H:2 * H])
        g_g = jnp.tanh(gates[:, 2 * H:3 * H])
        o_g = jax.nn.sigmoid(gates[:, 3 * H:4 * H])
        cs = f_g * cs + i_g * g_g
        hs = o_g * jnp.tanh(cs)
        q = hs
        s_mat = jnp.dot(q, h_t, preferred_element_type=_f32)
        sm = jnp.where(msk, s_mat, -1e30)
        m = jnp.max(sm, axis=1, keepdims=True)
        m = jnp.where(m > -1e29, m, 0.0)
        p = jnp.exp(jnp.where(msk, s_mat - m, -1e4))
        denom = jnp.sum(p, axis=1, keepdims=True)
        a = p / (denom + 1e-16)
        r = jnp.dot(a, h, preferred_element_type=_f32)
        q_star = jnp.concatenate([q, r], axis=1)
    o1 = jnp.maximum(
        jnp.dot(q_star, wo1_ref[...], preferred_element_type=_f32)
        + bo1_ref[...], 0.0)
    out_ref[...] = jnp.dot(o1, wo2_ref[...], preferred_element_type=_f32) \
        + bo2_ref[...]


def _s2s_call(agg, h, Wr, bc, batch_p, W_ihT, W_hhT, b_ih, b_hh,
              W_o1, b_o1, W_o2, b_o2):
    return pl.pallas_call(
        _s2s_body,
        out_shape=jax.ShapeDtypeStruct((B, 1), _f32),
    )(agg, h, Wr, bc, batch_p, W_ihT, W_hhT,
      b_ih.reshape(1, 4 * H), b_hh.reshape(1, 4 * H),
      W_o1, b_o1.reshape(1, H), W_o2, b_o2.reshape(1, 1))


_NBUF = 5


def _mp_sc_body(src_ref, et_ref, dst_ref, ht_ref, out_ref,
                srcv, etv, dstv, idxv, rows0, rows1, rows2, rows3, rows4,
                zbuf, aggsh, gs0, gs1, gs2, gs3, gs4,
                ss0, ss1, ss2, ss3, ss4):
    rows = (rows0, rows1, rows2, rows3, rows4)
    gsem = (gs0, gs1, gs2, gs3, gs4)
    ssem = (ss0, ss1, ss2, ss3, ss4)
    c = lax.axis_index("c")
    s = lax.axis_index("s")

    def pipeline(off, wch, nbuf):
        pltpu.async_copy(src_ref.at[pl.ds(off * CH, wch * CH)],
                         srcv.at[pl.ds(0, wch * CH)], gs0)
        pltpu.async_copy(et_ref.at[pl.ds(off * CH, wch * CH)],
                         etv.at[pl.ds(0, wch * CH)], gs1)
        pltpu.async_copy(dst_ref.at[pl.ds(off, wch)],
                         dstv.at[pl.ds(0, wch)], gs2)

        z16 = jnp.zeros((16,), _f32)

        def zrow(k, carry):
            for b in range(4):
                zbuf[k * 4 + b, pl.ds(0, 16)] = z16
                zbuf[k * 4 + b, pl.ds(16, 16)] = z16
            return carry

        lax.fori_loop(0, ZR // 4, zrow, 0)
        pltpu.sync_copy(zbuf, aggsh.at[pl.ds(s * ZR, ZR)])

        pltpu.make_async_copy(src_ref.at[pl.ds(off * CH, wch * CH)],
                              srcv.at[pl.ds(0, wch * CH)], gs0).wait()
        pltpu.make_async_copy(et_ref.at[pl.ds(off * CH, wch * CH)],
                              etv.at[pl.ds(0, wch * CH)], gs1).wait()
        pltpu.make_async_copy(dst_ref.at[pl.ds(off, wch)],
                              dstv.at[pl.ds(0, wch)], gs2).wait()

        def idxrow(k, carry):
            for j in range(8):
                sl = pl.ds(j * 16, 16)
                sv = srcv[pl.ds(k * CH + j * 16, 16)]
                ev = etv[pl.ds(k * CH + j * 16, 16)]
                idxv[k, sl] = ((sv >> 3) << 7) + ((ev >> 2) << 5) \
                    + ((sv & 7) << 2) + (ev & 3)
            return carry

        lax.fori_loop(0, wch, idxrow, 0)

        plsc.subcore_barrier()

        for b in range(nbuf):
            pltpu.async_copy(ht_ref.at[idxv.at[b]], rows[b], gsem[b])

        def outer(kk, carry):
            for b in range(nbuf):
                k = kk * nbuf + b
                pltpu.make_async_copy(ht_ref.at[idxv.at[k]], rows[b],
                                      gsem[b]).wait()
                pltpu.async_copy(rows[b], aggsh.at[dstv.at[k]], ssem[b],
                                 add=True)

                @pl.when(kk < wch // nbuf - 1)
                def _():
                    pltpu.make_async_copy(rows[b], aggsh.at[dstv.at[k]],
                                          ssem[b]).wait()
                    pltpu.async_copy(ht_ref.at[idxv.at[k + nbuf]], rows[b],
                                     gsem[b])
            return carry

        lax.fori_loop(0, wch // nbuf, outer, 0)
        for b in range(nbuf):
            k = wch - nbuf + b
            pltpu.make_async_copy(rows[b], aggsh.at[dstv.at[k]],
                                  ssem[b]).wait()

    w = c * 16 + s
    pipeline(w * WCH, WCH, _NBUF)
    plsc.subcore_barrier()

    pltpu.sync_copy(aggsh.at[pl.ds(s * ZR, ZR)], zbuf)
    pltpu.sync_copy(zbuf, out_ref.at[c, pl.ds(s * ZR, ZR)])


@functools.cache
def _mp_sc_call():
    mesh = plsc.VectorSubcoreMesh(core_axis_name="c", subcore_axis_name="s",
                                  num_cores=2, num_subcores=16)
    return pl.kernel(
        _mp_sc_body,
        out_type=jax.ShapeDtypeStruct((2, NROWS, H), _f32),
        mesh=mesh,
        compiler_params=pltpu.CompilerParams(use_tc_tiling_on_sc=False),
        scratch_types=[
            pltpu.VMEM((WCH * CH,), jnp.int32),
            pltpu.VMEM((WCH * CH,), jnp.int32),
            pltpu.VMEM((WCH, CH), jnp.int32),
            pltpu.VMEM((WCH, CH), jnp.int32),
            pltpu.VMEM((CH, H), _f32),
            pltpu.VMEM((CH, H), _f32),
            pltpu.VMEM((CH, H), _f32),
            pltpu.VMEM((CH, H), _f32),
            pltpu.VMEM((CH, H), _f32),
            pltpu.VMEM((ZR, H), _f32),
            pltpu.VMEM_SHARED((NROWS, H), _f32),
            pltpu.SemaphoreType.DMA,
            pltpu.SemaphoreType.DMA,
            pltpu.SemaphoreType.DMA,
            pltpu.SemaphoreType.DMA,
            pltpu.SemaphoreType.DMA,
            pltpu.SemaphoreType.DMA,
            pltpu.SemaphoreType.DMA,
            pltpu.SemaphoreType.DMA,
            pltpu.SemaphoreType.DMA,
            pltpu.SemaphoreType.DMA,
        ],
    )


def kernel(node_type, node_feat, edge_index, edge_type, batch,
           W_emb, b_emb, W_e1, b_e1, W_e2, b_e2, W_root, b_conv,
           W_ih, W_hh, b_ih, b_hh, W_o1, b_o1, W_o2, b_o2):
    i32 = jnp.int32
    nt_p = jnp.pad(node_type.astype(i32), (0, NP - N)).reshape(NP, 1)
    feat_p = jnp.pad(node_feat, ((0, NP - N), (0, H - FEAT)))
    W_f = jnp.pad(W_emb[NODE_TYPES:], ((0, H - FEAT), (0, 0)))
    batch_p = jnp.pad(batch.astype(i32), (0, NP - N),
                      constant_values=127).reshape(1, NP)

    pad_ar = jnp.arange(EP - E, dtype=i32)
    src1 = jnp.concatenate([edge_index[0].astype(i32), pad_ar % N])
    et1 = jnp.pad(edge_type.astype(i32), (0, EP - E))
    pad_dst = NP + (pad_ar % (NROWS - NP))
    dst2 = jnp.concatenate([edge_index[1].astype(i32),
                            pad_dst]).reshape(EROWS, CH)

    tflat = _tflat_call(W_e1, b_e1, W_e2, b_e2)
    tcat = tflat.reshape(16, H, H).transpose(1, 0, 2).reshape(H, 16 * H)

    h, htcat = _prep_call(nt_p, feat_p, W_emb, W_f, b_emb.reshape(1, H), tcat)
    for i in range(MP_ITER - 1):
        agg = _mp_sc_call()(src1, et1, dst2, htcat.reshape(NP * 16, H))
        h, htcat = _step_call(agg.reshape(2, NROWS * H // 128, 128), h,
                              W_root[i], b_conv[i].reshape(1, H), tcat)
    agg = _mp_sc_call()(src1, et1, dst2, htcat.reshape(NP * 16, H))

    return _s2s_call(agg, h, W_root[MP_ITER - 1],
                     b_conv[MP_ITER - 1].reshape(1, H), batch_p,
                     W_ih.T, W_hh.T, b_ih, b_hh, W_o1, b_o1, W_o2, b_o2)
